# baseline (speedup 1.0000x reference)
import jax
import jax.numpy as jnp
from jax import lax
from jax.experimental import pallas as pl
from jax.experimental.pallas import tpu as pltpu

N = 1024
V = 100000
C = 128
K = 8
G = N // K


def _loss_body(tgt_ref, *refs):
    xs = refs[:K]
    out_ref = refs[K]
    acc = refs[K + 1]
    t = pl.program_id(0)

    @pl.when(t == 0)
    def _():
        acc[...] = jnp.zeros((K, C), jnp.float32)

    lane = lax.broadcasted_iota(jnp.int32, (K, C), 1)
    row = lax.broadcasted_iota(jnp.int32, (K, C), 0)
    s = jnp.zeros((K, C), jnp.float32)
    for k in range(K):
        tv = tgt_ref[t * K + k]
        pick = jnp.where((row == k) & (lane == tv % C) & (tv > 0),
                         xs[k][...], 0.0)
        s = s + pick
    acc[...] = acc[...] + s

    @pl.when(t == G - 1)
    def _():
        out_ref[0, 0] = -jnp.sum(acc[...])


def _chunk_spec(k):
    return pl.BlockSpec(
        (K, C), lambda t, tgt: (t, tgt[t * K + k] // C))


@jax.jit
def kernel(input, target):
    tgt = target.reshape(-1).astype(jnp.int32)

    total = pl.pallas_call(
        _loss_body,
        grid_spec=pltpu.PrefetchScalarGridSpec(
            num_scalar_prefetch=1,
            grid=(G,),
            in_specs=[_chunk_spec(k) for k in range(K)],
            out_specs=pl.BlockSpec(
                (1, 1), lambda t, tgt: (0, 0), memory_space=pltpu.SMEM),
            scratch_shapes=[
                pltpu.VMEM((K, C), jnp.float32),
            ],
        ),
        out_shape=jax.ShapeDtypeStruct((1, 1), jnp.float32),
    )(tgt, *([input] * K))
    return total[0, 0]


# speedup vs baseline: 24.2713x; 24.2713x over previous
import jax
import jax.numpy as jnp
from jax import lax
from jax.experimental import pallas as pl
from jax.experimental.pallas import tpu as pltpu

N = 1024
V = 100000
C = 128
S = 8


def _loss_body(rowbase_ref, inpT_hbm, subsel_ref, out_ref, vals, sem):
    def _trip(t, carry):
        for k in range(64):
            i = t * 64 + k
            rb = pl.multiple_of(rowbase_ref[i], S)
            col = pl.multiple_of(jnp.bitwise_and(i, -128), C)
            pltpu.make_async_copy(
                inpT_hbm.at[pl.ds(rb, S), pl.ds(col, C)],
                vals.at[i],
                sem,
            ).start()
        return carry

    lax.fori_loop(0, N // 64, _trip, 0)

    def _drain(i, carry):
        pltpu.make_async_copy(
            inpT_hbm.at[pl.ds(0, S), pl.ds(0, C)], vals.at[0], sem
        ).wait()
        return carry

    lax.fori_loop(0, N, _drain, 0)

    row = lax.broadcasted_iota(jnp.int32, (N, S, C), 0)
    sub = lax.broadcasted_iota(jnp.int32, (N, S, C), 1)
    lane = lax.broadcasted_iota(jnp.int32, (N, S, C), 2)
    subsel = subsel_ref[...].reshape(N, 1, 1)
    cond = (sub == subsel) & (lane == jnp.bitwise_and(row, 127))
    picked = jnp.where(cond, vals[...], 0.0)
    out_ref[0, 0] = -jnp.sum(picked)


@jax.jit
def kernel(input, target):
    tgt = target.reshape(-1).astype(jnp.int32)
    rowbase = jnp.bitwise_and(tgt, -8)
    subsel = jnp.where(tgt > 0, jnp.bitwise_and(tgt, 7), -1).reshape(N, 1)

    total = pl.pallas_call(
        _loss_body,
        grid_spec=pltpu.PrefetchScalarGridSpec(
            num_scalar_prefetch=1,
            in_specs=[
                pl.BlockSpec(memory_space=pl.ANY),
                pl.BlockSpec(memory_space=pltpu.VMEM),
            ],
            out_specs=pl.BlockSpec(memory_space=pltpu.SMEM),
            scratch_shapes=[
                pltpu.VMEM((N, S, C), jnp.float32),
                pltpu.SemaphoreType.DMA,
            ],
        ),
        out_shape=jax.ShapeDtypeStruct((1, 1), jnp.float32),
    )(rowbase, input.T, subsel)
    return total[0, 0]


# speedup vs baseline: 32.9203x; 1.3563x over previous
import jax
import jax.numpy as jnp
from jax import lax
from jax.experimental import pallas as pl
from jax.experimental.pallas import tpu as pltpu

N = 1024
V = 100000
C = 128
S = 8


def _loss_body(rowbase_ref, inpT_hbm, subsel_ref, out_ref, vals, sem):
    def _trip(t, carry):
        for k in range(64):
            i = t * 64 + k
            rb = pl.multiple_of(rowbase_ref[i], S)
            col = pl.multiple_of(jnp.bitwise_and(i, -128), C)
            pltpu.make_async_copy(
                inpT_hbm.at[pl.ds(rb, S), pl.ds(col, C)],
                vals.at[i],
                sem,
            ).start()
        return carry

    lax.fori_loop(0, N // 64, _trip, 0)

    pltpu.make_async_copy(vals, vals, sem).wait()

    row = lax.broadcasted_iota(jnp.int32, (N, S, C), 0)
    sub = lax.broadcasted_iota(jnp.int32, (N, S, C), 1)
    lane = lax.broadcasted_iota(jnp.int32, (N, S, C), 2)
    subsel = subsel_ref[...].reshape(N, 1, 1)
    cond = (sub == subsel) & (lane == jnp.bitwise_and(row, 127))
    picked = jnp.where(cond, vals[...], 0.0)
    out_ref[0, 0] = -jnp.sum(picked)


@jax.jit
def kernel(input, target):
    tgt = target.reshape(-1).astype(jnp.int32)
    rowbase = jnp.bitwise_and(tgt, -8)
    subsel = jnp.where(tgt > 0, jnp.bitwise_and(tgt, 7), -1).reshape(N, 1)

    total = pl.pallas_call(
        _loss_body,
        grid_spec=pltpu.PrefetchScalarGridSpec(
            num_scalar_prefetch=1,
            in_specs=[
                pl.BlockSpec(memory_space=pl.ANY),
                pl.BlockSpec(memory_space=pltpu.VMEM),
            ],
            out_specs=pl.BlockSpec(memory_space=pltpu.SMEM),
            scratch_shapes=[
                pltpu.VMEM((N, S, C), jnp.float32),
                pltpu.SemaphoreType.DMA,
            ],
        ),
        out_shape=jax.ShapeDtypeStruct((1, 1), jnp.float32),
    )(rowbase, input.T, subsel)
    return total[0, 0]


# speedup vs baseline: 40.1220x; 1.2188x over previous
import jax
import jax.numpy as jnp
from jax import lax
from jax.experimental import pallas as pl
from jax.experimental.pallas import tpu as pltpu

N = 1024
V = 100000
C = 128


def _loss_body(tgt_ref, inpT_hbm, lanesel_ref, out_ref, vals, sem):
    def _trip(t, carry):
        for k in range(64):
            i = t * 64 + k
            tv = tgt_ref[i]
            col = pl.multiple_of(jnp.bitwise_and(i, -128), C)
            pltpu.make_async_copy(
                inpT_hbm.at[pl.ds(tv, 1), pl.ds(col, C)],
                vals.at[pl.ds(i, 1), :],
                sem,
            ).start()
        return carry

    lax.fori_loop(0, N // 64, _trip, 0)
    pltpu.make_async_copy(vals, vals, sem).wait()

    lane = lax.broadcasted_iota(jnp.int32, (N, C), 1)
    picked = jnp.where(lane == lanesel_ref[...], vals[...], 0.0)
    out_ref[0, 0] = -jnp.sum(picked)


@jax.jit
def kernel(input, target):
    tgt = target.reshape(-1).astype(jnp.int32)
    lanesel = jnp.where(
        tgt > 0, jnp.bitwise_and(jnp.arange(N, dtype=jnp.int32), 127), -1
    ).reshape(N, 1)

    total = pl.pallas_call(
        _loss_body,
        grid_spec=pltpu.PrefetchScalarGridSpec(
            num_scalar_prefetch=1,
            in_specs=[
                pl.BlockSpec(memory_space=pl.ANY),
                pl.BlockSpec(memory_space=pltpu.VMEM),
            ],
            out_specs=pl.BlockSpec(memory_space=pltpu.SMEM),
            scratch_shapes=[
                pltpu.VMEM((N, C), jnp.float32),
                pltpu.SemaphoreType.DMA,
            ],
        ),
        out_shape=jax.ShapeDtypeStruct((1, 1), jnp.float32),
    )(tgt, input.T, lanesel)
    return total[0, 0]


# speedup vs baseline: 43.6703x; 1.0884x over previous
import jax
import jax.numpy as jnp
from jax import lax
from jax.experimental import pallas as pl
from jax.experimental.pallas import tpu as pltpu

N = 1024
V = 100000
C = 128


def _loss_body(tgt_ref, inpT_hbm, lanesel_ref, out_ref, vals, sem):
    def _trip(t, carry):
        for k in range(64):
            i = t * 64 + k
            tv = tgt_ref[i]
            col = pl.multiple_of(jnp.bitwise_and(i, -128), C)
            pltpu.make_async_copy(
                inpT_hbm.at[pl.ds(tv, 1), pl.ds(col, C)],
                vals.at[pl.ds(i, 1), :],
                sem,
            ).start()
        return carry

    lax.fori_loop(0, N // 64, _trip, 0)
    pltpu.make_async_copy(vals, vals, sem).wait()

    lane = lax.broadcasted_iota(jnp.int32, (N // C, C, C), 2)
    sel = lanesel_ref[...].reshape(N // C, C, 1)
    picked = jnp.where(lane == sel, vals[...].reshape(N // C, C, C), 0.0)
    out_ref[0, 0] = -jnp.sum(picked)


@jax.jit
def kernel(input, target):
    tgt = target.reshape(-1).astype(jnp.int32)
    lanesel = jnp.where(
        tgt > 0, jnp.bitwise_and(jnp.arange(N, dtype=jnp.int32), 127), -1
    ).reshape(N // C, C)

    total = pl.pallas_call(
        _loss_body,
        grid_spec=pltpu.PrefetchScalarGridSpec(
            num_scalar_prefetch=1,
            in_specs=[
                pl.BlockSpec(memory_space=pl.ANY),
                pl.BlockSpec(memory_space=pltpu.VMEM),
            ],
            out_specs=pl.BlockSpec(memory_space=pltpu.SMEM),
            scratch_shapes=[
                pltpu.VMEM((N, C), jnp.float32),
                pltpu.SemaphoreType.DMA,
            ],
        ),
        out_shape=jax.ShapeDtypeStruct((1, 1), jnp.float32),
    )(tgt, input.T, lanesel)
    return total[0, 0]
